# baseline (speedup 1.0000x reference)
import functools

import jax
import jax.numpy as jnp
from jax import lax
from jax.experimental import pallas as pl
from jax.experimental.pallas import tpu as pltpu
from jax.experimental.pallas import tpu_sc as plsc

_K = 197
_EKP_COL = 200
_DISC_COL = 400
_D = 512
_NW = 32
_CH = 64


def _sig(x):
    return 0.5 * jnp.tanh(0.5 * x) + 0.5


def _prep_body(kdT_ref, ekpT_ref, discT_ref, sembT_ref, tbl_ref, ssembT_ref):
    n = kdT_ref.shape[1]
    tbl_ref[:, 0:_K] = jnp.transpose(_sig(kdT_ref[...]))
    tbl_ref[:, _K:_EKP_COL] = jnp.zeros((n, _EKP_COL - _K), jnp.float32)
    tbl_ref[:, _EKP_COL:_EKP_COL + _K] = jnp.transpose(ekpT_ref[...])
    tbl_ref[:, _EKP_COL + _K:_D] = jnp.zeros((n, _D - _EKP_COL - _K),
                                             jnp.float32)
    tbl_ref[:, _DISC_COL:_DISC_COL + 1] = jnp.transpose(
        10.0 * _sig(discT_ref[...]))
    ssembT_ref[...] = _sig(sembT_ref[...])


def _prep(kdT, ekpT, discT, sembT):
    k, exer_n = kdT.shape
    stu_n = sembT.shape[1]
    return pl.pallas_call(
        _prep_body,
        out_shape=[
            jax.ShapeDtypeStruct((exer_n, _D), jnp.float32),
            jax.ShapeDtypeStruct((k, stu_n), jnp.float32),
        ],
    )(kdT, ekpT, discT, sembT)


def _sc_gather(tbl, idx):
    B = idx.shape[0]
    D = tbl.shape[1]
    bpw = B // _NW
    mesh = plsc.VectorSubcoreMesh(core_axis_name="c", subcore_axis_name="s")

    nch = bpw // _CH

    @functools.partial(
        pl.kernel,
        mesh=mesh,
        out_type=jax.ShapeDtypeStruct((B, D), jnp.float32),
        scratch_types=[
            pltpu.VMEM((_CH,), jnp.int32),
            pltpu.VMEM((_CH,), jnp.int32),
            pltpu.VMEM((_CH, D), jnp.float32),
            pltpu.VMEM((_CH, D), jnp.float32),
            pltpu.SemaphoreType.DMA,
            pltpu.SemaphoreType.DMA,
            pltpu.SemaphoreType.DMA,
            pltpu.SemaphoreType.DMA,
        ],
    )
    def k(tbl_hbm, idx_hbm, out_hbm, idx_v0, idx_v1, r0, r1,
          g0, g1, s0, s1):
        wid = lax.axis_index("s") * 2 + lax.axis_index("c")
        base = wid * bpw
        idx_v = [idx_v0, idx_v1]
        rows = [r0, r1]
        gsem = [g0, g1]
        ssem = [s0, s1]
        gh = [None, None]
        sh = [None, None]
        for b in range(min(2, nch)):
            pltpu.sync_copy(idx_hbm.at[pl.ds(base + b * _CH, _CH)], idx_v[b])
            gh[b] = pltpu.async_copy(tbl_hbm.at[idx_v[b]], rows[b], gsem[b])
        for ci in range(nch):
            b = ci % 2
            gh[b].wait()
            sh[b] = pltpu.async_copy(
                rows[b], out_hbm.at[pl.ds(base + ci * _CH, _CH)], ssem[b])
            if ci + 2 < nch:
                sh[b].wait()
                pltpu.sync_copy(
                    idx_hbm.at[pl.ds(base + (ci + 2) * _CH, _CH)], idx_v[b])
                gh[b] = pltpu.async_copy(tbl_hbm.at[idx_v[b]], rows[b],
                                         gsem[b])
        for b in range(min(2, nch)):
            if sh[b] is not None:
                sh[b].wait()

    return k(tbl, idx)


def _mlp_body(g_ref, mT_ref, sid_ref, ssembT_ref,
              w1_ref, b1_ref, w2_ref, b2_ref, w3_ref, b3_ref,
              outT_ref, ekpT_ref):
    bb = g_ref.shape[0]
    stu_n = ssembT_ref.shape[1]
    ids = jnp.reshape(sid_ref[...], (1, bb))
    col = lax.broadcasted_iota(jnp.int32, (stu_n, bb), 0)
    ohT = (ids == col).astype(jnp.float32)
    statT = jnp.dot(ssembT_ref[...], ohT,
                    preferred_element_type=jnp.float32)

    gT = jnp.transpose(g_ref[...])
    skdT = gT[0:_K, :]
    ekpT = gT[_EKP_COL:_EKP_COL + _K, :]
    ekpT_ref[...] = ekpT
    discT = gT[_DISC_COL:_DISC_COL + 1, :]

    xT = discT * (statT - skdT) * (mT_ref[...] * _sig(ekpT))
    h1T = _sig(
        jnp.dot(w1_ref[...].astype(jnp.bfloat16), xT.astype(jnp.bfloat16),
                preferred_element_type=jnp.float32)
        + b1_ref[...])
    h2T = _sig(
        jnp.dot(w2_ref[...].astype(jnp.bfloat16), h1T.astype(jnp.bfloat16),
                preferred_element_type=jnp.float32)
        + b2_ref[...])
    pT = _sig(
        jnp.dot(w3_ref[...], h2T, preferred_element_type=jnp.float32)
        + b3_ref[...])
    outT_ref[0:1, :] = 1.0 - pT
    outT_ref[1:2, :] = pT


def _tc_mlp(gathered, masksT, sid, ssembT, w1, b1c, w2, b2c, w3, b3c,
            half_blocks, half):
    BB = 1024
    H = half_blocks * BB
    off = half * half_blocks
    k, stu_n = ssembT.shape
    l1, l2 = w1.shape[0], w2.shape[0]
    full = lambda shp: pl.BlockSpec(shp, lambda i: (0, 0))
    return pl.pallas_call(
        _mlp_body,
        grid=(half_blocks,),
        in_specs=[
            pl.BlockSpec((BB, _D), lambda i: (i, 0)),
            pl.BlockSpec((k, BB), lambda i: (0, i + off)),
            pl.BlockSpec((BB,), lambda i: (i + off,)),
            full((k, stu_n)),
            full((l1, k)), full((l1, 1)),
            full((l2, l1)), full((l2, 1)),
            full((1, l2)), full((1, 1)),
        ],
        out_specs=[
            pl.BlockSpec((2, BB), lambda i: (0, i)),
            pl.BlockSpec((k, BB), lambda i: (0, i)),
        ],
        out_shape=[
            jax.ShapeDtypeStruct((2, H), jnp.float32),
            jax.ShapeDtypeStruct((k, H), jnp.float32),
        ],
    )(gathered, masksT, sid, ssembT, w1, b1c, w2, b2c, w3, b3c)


def kernel(stu_id, input_exercise, knowledge_masks, student_emb, k_difficulty,
           e_discrimination, e_k_prob, W1, b1, W2, b2, W3, b3):
    tbl, ssembT = _prep(k_difficulty.T, e_k_prob.T, e_discrimination.T,
                        student_emb.T)
    idx = input_exercise.astype(jnp.int32)
    B = idx.shape[0]
    H = B // 2
    sid = stu_id.astype(jnp.int32)
    masksT = knowledge_masks.T
    args = (ssembT, W1, b1.reshape(-1, 1), W2, b2.reshape(-1, 1),
            W3, b3.reshape(-1, 1))
    g_a = _sc_gather(tbl, idx[:H])
    g_b = _sc_gather(tbl, idx[H:])
    outT_a, ekpT_a = _tc_mlp(g_a, masksT, sid, *args,
                             half_blocks=H // 1024, half=0)
    outT_b, ekpT_b = _tc_mlp(g_b, masksT, sid, *args,
                             half_blocks=H // 1024, half=1)
    outT = jnp.concatenate([outT_a, outT_b], axis=1)
    ekpT = jnp.concatenate([ekpT_a, ekpT_b], axis=1)
    return (outT.T, ekpT.T)


# speedup vs baseline: 1.2454x; 1.2454x over previous
import functools

import jax
import jax.numpy as jnp
from jax import lax
from jax.experimental import pallas as pl
from jax.experimental.pallas import tpu as pltpu
from jax.experimental.pallas import tpu_sc as plsc

_K = 197
_KP2 = 99
_SKD_COL = 200
_DISC_COL = 304
_D = 384
_NW = 32
_CH = 64


def _sig(x):
    return 0.5 * jnp.tanh(0.5 * x) + 0.5


def _prep_body(kdT_ref, ekpT_ref, discT_ref, sembT_ref, tbl_ref, ssembT_ref):
    n = kdT_ref.shape[1]
    tbl_ref[:, 0:_K] = jnp.transpose(ekpT_ref[...])
    skd = jnp.transpose(_sig(kdT_ref[...])).astype(jnp.bfloat16)
    skd = jnp.concatenate(
        [skd, jnp.zeros((n, 2 * _KP2 - _K), jnp.bfloat16)], axis=1)
    u = jax.lax.bitcast_convert_type(skd, jnp.uint16).astype(jnp.uint32)
    w = u[:, 0:_KP2] | (u[:, _KP2:2 * _KP2] << 16)
    tbl_ref[:, _SKD_COL:_SKD_COL + _KP2] = jax.lax.bitcast_convert_type(
        w, jnp.float32)
    tbl_ref[:, _DISC_COL:_DISC_COL + 1] = jnp.transpose(
        10.0 * _sig(discT_ref[...]))
    ssembT_ref[...] = _sig(sembT_ref[...])


def _prep(kdT, ekpT, discT, sembT):
    k, exer_n = kdT.shape
    stu_n = sembT.shape[1]
    return pl.pallas_call(
        _prep_body,
        out_shape=[
            jax.ShapeDtypeStruct((exer_n, _D), jnp.float32),
            jax.ShapeDtypeStruct((k, stu_n), jnp.float32),
        ],
    )(kdT, ekpT, discT, sembT)


def _sc_gather(tbl, idx):
    B = idx.shape[0]
    D = tbl.shape[1]
    bpw = B // _NW
    mesh = plsc.VectorSubcoreMesh(core_axis_name="c", subcore_axis_name="s")

    nch = bpw // _CH

    @functools.partial(
        pl.kernel,
        mesh=mesh,
        out_type=jax.ShapeDtypeStruct((B, D), jnp.float32),
        scratch_types=[
            pltpu.VMEM((_CH,), jnp.int32),
            pltpu.VMEM((_CH,), jnp.int32),
            pltpu.VMEM((_CH, D), jnp.float32),
            pltpu.VMEM((_CH, D), jnp.float32),
            pltpu.SemaphoreType.DMA,
            pltpu.SemaphoreType.DMA,
            pltpu.SemaphoreType.DMA,
            pltpu.SemaphoreType.DMA,
        ],
    )
    def k(tbl_hbm, idx_hbm, out_hbm, idx_v0, idx_v1, r0, r1,
          g0, g1, s0, s1):
        wid = lax.axis_index("s") * 2 + lax.axis_index("c")
        base = wid * bpw
        idx_v = [idx_v0, idx_v1]
        rows = [r0, r1]
        gsem = [g0, g1]
        ssem = [s0, s1]
        gh = [None, None]
        sh = [None, None]
        for b in range(min(2, nch)):
            pltpu.sync_copy(idx_hbm.at[pl.ds(base + b * _CH, _CH)], idx_v[b])
            gh[b] = pltpu.async_copy(tbl_hbm.at[idx_v[b]], rows[b], gsem[b])
        for ci in range(nch):
            b = ci % 2
            gh[b].wait()
            sh[b] = pltpu.async_copy(
                rows[b], out_hbm.at[pl.ds(base + ci * _CH, _CH)], ssem[b])
            if ci + 2 < nch:
                sh[b].wait()
                pltpu.sync_copy(
                    idx_hbm.at[pl.ds(base + (ci + 2) * _CH, _CH)], idx_v[b])
                gh[b] = pltpu.async_copy(tbl_hbm.at[idx_v[b]], rows[b],
                                         gsem[b])
        for b in range(min(2, nch)):
            if sh[b] is not None:
                sh[b].wait()

    return k(tbl, idx)


def _mlp_body(g_ref, mT_ref, sid_ref, ssembT_ref,
              w1_ref, b1_ref, w2_ref, b2_ref, w3_ref, b3_ref,
              outT_ref, ekpT_ref):
    bb = g_ref.shape[0]
    stu_n = ssembT_ref.shape[1]
    ids = jnp.reshape(sid_ref[...], (1, bb))
    col = lax.broadcasted_iota(jnp.int32, (stu_n, bb), 0)
    ohT = (ids == col).astype(jnp.float32)
    statT = jnp.dot(ssembT_ref[...], ohT,
                    preferred_element_type=jnp.float32)

    gT = jnp.transpose(g_ref[...])
    ekpT = gT[0:_K, :]
    ekpT_ref[...] = ekpT
    discT = gT[_DISC_COL:_DISC_COL + 1, :]
    w = jax.lax.bitcast_convert_type(
        gT[_SKD_COL:_SKD_COL + _KP2, :], jnp.uint32)
    lo = jax.lax.bitcast_convert_type(
        (w & 0xFFFF).astype(jnp.uint16), jnp.bfloat16)
    hi = jax.lax.bitcast_convert_type(
        (w >> 16).astype(jnp.uint16), jnp.bfloat16)
    skdT = jnp.concatenate([lo, hi], axis=0)[0:_K, :]

    bf = jnp.bfloat16
    xT = (discT.astype(bf) * (statT.astype(bf) - skdT)
          * (mT_ref[...].astype(bf) * _sig(ekpT).astype(bf)))
    h1T = _sig(
        jnp.dot(w1_ref[...].astype(jnp.bfloat16), xT,
                preferred_element_type=jnp.float32)
        + b1_ref[...])
    h2T = _sig(
        jnp.dot(w2_ref[...].astype(jnp.bfloat16), h1T.astype(jnp.bfloat16),
                preferred_element_type=jnp.float32)
        + b2_ref[...])
    pT = _sig(
        jnp.dot(w3_ref[...], h2T, preferred_element_type=jnp.float32)
        + b3_ref[...])
    outT_ref[0:1, :] = 1.0 - pT
    outT_ref[1:2, :] = pT


def _tc_mlp(gathered, masksT, sid, ssembT, w1, b1c, w2, b2c, w3, b3c,
            half_blocks, half):
    BB = 1024
    H = half_blocks * BB
    off = half * half_blocks
    k, stu_n = ssembT.shape
    l1, l2 = w1.shape[0], w2.shape[0]
    full = lambda shp: pl.BlockSpec(shp, lambda i: (0, 0))
    return pl.pallas_call(
        _mlp_body,
        grid=(half_blocks,),
        in_specs=[
            pl.BlockSpec((BB, _D), lambda i: (i, 0)),
            pl.BlockSpec((k, BB), lambda i: (0, i + off)),
            pl.BlockSpec((BB,), lambda i: (i + off,)),
            full((k, stu_n)),
            full((l1, k)), full((l1, 1)),
            full((l2, l1)), full((l2, 1)),
            full((1, l2)), full((1, 1)),
        ],
        out_specs=[
            pl.BlockSpec((2, BB), lambda i: (0, i)),
            pl.BlockSpec((k, BB), lambda i: (0, i)),
        ],
        out_shape=[
            jax.ShapeDtypeStruct((2, H), jnp.float32),
            jax.ShapeDtypeStruct((k, H), jnp.float32),
        ],
    )(gathered, masksT, sid, ssembT, w1, b1c, w2, b2c, w3, b3c)


def kernel(stu_id, input_exercise, knowledge_masks, student_emb, k_difficulty,
           e_discrimination, e_k_prob, W1, b1, W2, b2, W3, b3):
    tbl, ssembT = _prep(k_difficulty.T, e_k_prob.T, e_discrimination.T,
                        student_emb.T)
    idx = input_exercise.astype(jnp.int32)
    gathered = _sc_gather(tbl, idx)
    outT, ekpT = _tc_mlp(
        gathered, knowledge_masks.T, stu_id.astype(jnp.int32),
        ssembT, W1, b1.reshape(-1, 1), W2, b2.reshape(-1, 1),
        W3, b3.reshape(-1, 1), half_blocks=idx.shape[0] // 1024, half=0)
    return (outT.T, ekpT.T)


# speedup vs baseline: 1.3192x; 1.0592x over previous
import functools

import jax
import jax.numpy as jnp
from jax import lax
from jax.experimental import pallas as pl
from jax.experimental.pallas import tpu as pltpu
from jax.experimental.pallas import tpu_sc as plsc

_K = 197
_KP2 = 99
_SKD_COL = 200
_DISC_COL = 304
_D = 384
_NW = 32
_CH = 64


def _sig(x):
    return 0.5 * jnp.tanh(0.5 * x) + 0.5


def _prep_body(kdT_ref, ekpT_ref, discT_ref, sembT_ref, tbl_ref, ssembT_ref):
    n = kdT_ref.shape[1]
    tbl_ref[:, 0:_K] = jnp.transpose(ekpT_ref[...])
    skd = jnp.transpose(_sig(kdT_ref[...])).astype(jnp.bfloat16)
    skd = jnp.concatenate(
        [skd, jnp.zeros((n, 2 * _KP2 - _K), jnp.bfloat16)], axis=1)
    u = jax.lax.bitcast_convert_type(skd, jnp.uint16).astype(jnp.uint32)
    w = u[:, 0:_KP2] | (u[:, _KP2:2 * _KP2] << 16)
    tbl_ref[:, _SKD_COL:_SKD_COL + _KP2] = jax.lax.bitcast_convert_type(
        w, jnp.float32)
    tbl_ref[:, _DISC_COL:_DISC_COL + 1] = jnp.transpose(
        10.0 * _sig(discT_ref[...]))
    ssembT_ref[...] = _sig(sembT_ref[...])


def _prep(kdT, ekpT, discT, sembT):
    k, exer_n = kdT.shape
    stu_n = sembT.shape[1]
    nb = 1024
    grid = ((exer_n + nb - 1) // nb,)
    return pl.pallas_call(
        _prep_body,
        grid=grid,
        in_specs=[
            pl.BlockSpec((k, nb), lambda i: (0, i)),
            pl.BlockSpec((k, nb), lambda i: (0, i)),
            pl.BlockSpec((1, nb), lambda i: (0, i)),
            pl.BlockSpec((k, stu_n), lambda i: (0, 0)),
        ],
        out_specs=[
            pl.BlockSpec((nb, _D), lambda i: (i, 0)),
            pl.BlockSpec((k, stu_n), lambda i: (0, 0)),
        ],
        out_shape=[
            jax.ShapeDtypeStruct((exer_n, _D), jnp.float32),
            jax.ShapeDtypeStruct((k, stu_n), jnp.float32),
        ],
    )(kdT, ekpT, discT, sembT)


def _sc_gather(tbl, idx):
    B = idx.shape[0]
    D = tbl.shape[1]
    bpw = B // _NW
    mesh = plsc.VectorSubcoreMesh(core_axis_name="c", subcore_axis_name="s")

    nch = bpw // _CH

    @functools.partial(
        pl.kernel,
        mesh=mesh,
        out_type=jax.ShapeDtypeStruct((B, D), jnp.float32),
        scratch_types=[
            pltpu.VMEM((_CH,), jnp.int32),
            pltpu.VMEM((_CH,), jnp.int32),
            pltpu.VMEM((_CH, D), jnp.float32),
            pltpu.VMEM((_CH, D), jnp.float32),
            pltpu.SemaphoreType.DMA,
            pltpu.SemaphoreType.DMA,
            pltpu.SemaphoreType.DMA,
            pltpu.SemaphoreType.DMA,
        ],
    )
    def k(tbl_hbm, idx_hbm, out_hbm, idx_v0, idx_v1, r0, r1,
          g0, g1, s0, s1):
        wid = lax.axis_index("s") * 2 + lax.axis_index("c")
        base = wid * bpw
        idx_v = [idx_v0, idx_v1]
        rows = [r0, r1]
        gsem = [g0, g1]
        ssem = [s0, s1]
        gh = [None, None]
        sh = [None, None]
        for b in range(min(2, nch)):
            pltpu.sync_copy(idx_hbm.at[pl.ds(base + b * _CH, _CH)], idx_v[b])
            gh[b] = pltpu.async_copy(tbl_hbm.at[idx_v[b]], rows[b], gsem[b])
        for ci in range(nch):
            b = ci % 2
            gh[b].wait()
            sh[b] = pltpu.async_copy(
                rows[b], out_hbm.at[pl.ds(base + ci * _CH, _CH)], ssem[b])
            if ci + 2 < nch:
                sh[b].wait()
                pltpu.sync_copy(
                    idx_hbm.at[pl.ds(base + (ci + 2) * _CH, _CH)], idx_v[b])
                gh[b] = pltpu.async_copy(tbl_hbm.at[idx_v[b]], rows[b],
                                         gsem[b])
        for b in range(min(2, nch)):
            if sh[b] is not None:
                sh[b].wait()

    return k(tbl, idx)


def _mlp_body(g_ref, mT_ref, sid_ref, ssembT_ref,
              w1_ref, b1_ref, w2_ref, b2_ref, w3_ref, b3_ref,
              outT_ref, ekpT_ref):
    bb = g_ref.shape[0]
    stu_n = ssembT_ref.shape[1]
    ids = jnp.reshape(sid_ref[...], (1, bb))
    col = lax.broadcasted_iota(jnp.int32, (stu_n, bb), 0)
    ohT = (ids == col).astype(jnp.float32)
    statT = jnp.dot(ssembT_ref[...], ohT,
                    preferred_element_type=jnp.float32)

    gT = jnp.transpose(g_ref[...])
    ekpT = gT[0:_K, :]
    ekpT_ref[...] = ekpT
    discT = gT[_DISC_COL:_DISC_COL + 1, :]
    w = jax.lax.bitcast_convert_type(
        gT[_SKD_COL:_SKD_COL + _KP2, :], jnp.uint32)
    lo = jax.lax.bitcast_convert_type(
        (w & 0xFFFF).astype(jnp.uint16), jnp.bfloat16)
    hi = jax.lax.bitcast_convert_type(
        (w >> 16).astype(jnp.uint16), jnp.bfloat16)
    skdT = jnp.concatenate([lo, hi], axis=0)[0:_K, :]

    bf = jnp.bfloat16
    xT = (discT.astype(bf) * (statT.astype(bf) - skdT)
          * (mT_ref[...].astype(bf) * _sig(ekpT).astype(bf)))
    h1T = _sig(
        jnp.dot(w1_ref[...].astype(jnp.bfloat16), xT,
                preferred_element_type=jnp.float32)
        + b1_ref[...])
    h2T = _sig(
        jnp.dot(w2_ref[...].astype(jnp.bfloat16), h1T.astype(jnp.bfloat16),
                preferred_element_type=jnp.float32)
        + b2_ref[...])
    pT = _sig(
        jnp.dot(w3_ref[...], h2T, preferred_element_type=jnp.float32)
        + b3_ref[...])
    outT_ref[0:1, :] = 1.0 - pT
    outT_ref[1:2, :] = pT


def _tc_mlp(gathered, masksT, sid, ssembT, w1, b1c, w2, b2c, w3, b3c,
            half_blocks, half):
    BB = 2048
    H = half_blocks * BB
    off = half * half_blocks
    k, stu_n = ssembT.shape
    l1, l2 = w1.shape[0], w2.shape[0]
    full = lambda shp: pl.BlockSpec(shp, lambda i: (0, 0))
    return pl.pallas_call(
        _mlp_body,
        grid=(half_blocks,),
        in_specs=[
            pl.BlockSpec((BB, _D), lambda i: (i, 0)),
            pl.BlockSpec((k, BB), lambda i: (0, i + off)),
            pl.BlockSpec((BB,), lambda i: (i + off,)),
            full((k, stu_n)),
            full((l1, k)), full((l1, 1)),
            full((l2, l1)), full((l2, 1)),
            full((1, l2)), full((1, 1)),
        ],
        out_specs=[
            pl.BlockSpec((2, BB), lambda i: (0, i)),
            pl.BlockSpec((k, BB), lambda i: (0, i)),
        ],
        out_shape=[
            jax.ShapeDtypeStruct((2, H), jnp.float32),
            jax.ShapeDtypeStruct((k, H), jnp.float32),
        ],
    )(gathered, masksT, sid, ssembT, w1, b1c, w2, b2c, w3, b3c)


def kernel(stu_id, input_exercise, knowledge_masks, student_emb, k_difficulty,
           e_discrimination, e_k_prob, W1, b1, W2, b2, W3, b3):
    tbl, ssembT = _prep(k_difficulty.T, e_k_prob.T, e_discrimination.T,
                        student_emb.T)
    idx = input_exercise.astype(jnp.int32)
    gathered = _sc_gather(tbl, idx)
    outT, ekpT = _tc_mlp(
        gathered, knowledge_masks.T, stu_id.astype(jnp.int32),
        ssembT, W1, b1.reshape(-1, 1), W2, b2.reshape(-1, 1),
        W3, b3.reshape(-1, 1), half_blocks=idx.shape[0] // 2048, half=0)
    return (outT.T, ekpT.T)


# speedup vs baseline: 1.3219x; 1.0021x over previous
import functools

import jax
import jax.numpy as jnp
from jax import lax
from jax.experimental import pallas as pl
from jax.experimental.pallas import tpu as pltpu
from jax.experimental.pallas import tpu_sc as plsc

_K = 197
_KP2 = 99
_SKD_COL = 200
_DISC_COL = 304
_D = 384
_NW = 32
_CH = 64


def _sig(x):
    return 0.5 * jnp.tanh(0.5 * x) + 0.5


def _prep_body(kdT_ref, ekpT_ref, discT_ref, sembT_ref, tbl_ref, ssembT_ref):
    n = kdT_ref.shape[1]
    tbl_ref[:, 0:_K] = jnp.transpose(ekpT_ref[...])
    skd = jnp.transpose(_sig(kdT_ref[...])).astype(jnp.bfloat16)
    skd = jnp.concatenate(
        [skd, jnp.zeros((n, 2 * _KP2 - _K), jnp.bfloat16)], axis=1)
    u = jax.lax.bitcast_convert_type(skd, jnp.uint16).astype(jnp.uint32)
    w = u[:, 0:_KP2] | (u[:, _KP2:2 * _KP2] << 16)
    tbl_ref[:, _SKD_COL:_SKD_COL + _KP2] = jax.lax.bitcast_convert_type(
        w, jnp.float32)
    tbl_ref[:, _DISC_COL:_DISC_COL + 1] = jnp.transpose(
        10.0 * _sig(discT_ref[...]))
    ssembT_ref[...] = _sig(sembT_ref[...])


def _prep(kdT, ekpT, discT, sembT):
    k, exer_n = kdT.shape
    stu_n = sembT.shape[1]
    nb = 1024
    grid = ((exer_n + nb - 1) // nb,)
    return pl.pallas_call(
        _prep_body,
        grid=grid,
        in_specs=[
            pl.BlockSpec((k, nb), lambda i: (0, i)),
            pl.BlockSpec((k, nb), lambda i: (0, i)),
            pl.BlockSpec((1, nb), lambda i: (0, i)),
            pl.BlockSpec((k, stu_n), lambda i: (0, 0)),
        ],
        out_specs=[
            pl.BlockSpec((nb, _D), lambda i: (i, 0)),
            pl.BlockSpec((k, stu_n), lambda i: (0, 0)),
        ],
        out_shape=[
            jax.ShapeDtypeStruct((exer_n, _D), jnp.float32),
            jax.ShapeDtypeStruct((k, stu_n), jnp.float32),
        ],
    )(kdT, ekpT, discT, sembT)


def _sc_gather(tbl, idx):
    B = idx.shape[0]
    D = tbl.shape[1]
    bpw = B // _NW
    mesh = plsc.VectorSubcoreMesh(core_axis_name="c", subcore_axis_name="s")

    nch = bpw // _CH

    @functools.partial(
        pl.kernel,
        mesh=mesh,
        out_type=jax.ShapeDtypeStruct((B, D), jnp.float32),
        scratch_types=[
            pltpu.VMEM((_CH,), jnp.int32),
            pltpu.VMEM((_CH,), jnp.int32),
            pltpu.VMEM((_CH,), jnp.int32),
            pltpu.VMEM((_CH, D), jnp.float32),
            pltpu.VMEM((_CH, D), jnp.float32),
            pltpu.VMEM((_CH, D), jnp.float32),
            pltpu.SemaphoreType.DMA,
            pltpu.SemaphoreType.DMA,
            pltpu.SemaphoreType.DMA,
            pltpu.SemaphoreType.DMA,
            pltpu.SemaphoreType.DMA,
            pltpu.SemaphoreType.DMA,
        ],
    )
    def k(tbl_hbm, idx_hbm, out_hbm, idx_v0, idx_v1, idx_v2, r0, r1, r2,
          g0, g1, g2, s0, s1, s2):
        wid = lax.axis_index("s") * 2 + lax.axis_index("c")
        base = wid * bpw
        nb = 3
        idx_v = [idx_v0, idx_v1, idx_v2]
        rows = [r0, r1, r2]
        gsem = [g0, g1, g2]
        ssem = [s0, s1, s2]
        gh = [None] * nb
        sh = [None] * nb
        for b in range(min(nb, nch)):
            pltpu.sync_copy(idx_hbm.at[pl.ds(base + b * _CH, _CH)], idx_v[b])
            gh[b] = pltpu.async_copy(tbl_hbm.at[idx_v[b]], rows[b], gsem[b])
        for ci in range(nch):
            b = ci % nb
            gh[b].wait()
            sh[b] = pltpu.async_copy(
                rows[b], out_hbm.at[pl.ds(base + ci * _CH, _CH)], ssem[b])
            if ci + nb < nch:
                sh[b].wait()
                pltpu.sync_copy(
                    idx_hbm.at[pl.ds(base + (ci + nb) * _CH, _CH)], idx_v[b])
                gh[b] = pltpu.async_copy(tbl_hbm.at[idx_v[b]], rows[b],
                                         gsem[b])
        for b in range(min(nb, nch)):
            if sh[b] is not None:
                sh[b].wait()

    return k(tbl, idx)


def _mlp_body(g_ref, mT_ref, sid_ref, ssembT_ref,
              w1_ref, b1_ref, w2_ref, b2_ref, w3_ref, b3_ref,
              outT_ref, ekpT_ref):
    bb = g_ref.shape[0]
    stu_n = ssembT_ref.shape[1]
    ids = jnp.reshape(sid_ref[...], (1, bb))
    col = lax.broadcasted_iota(jnp.int32, (stu_n, bb), 0)
    ohT = (ids == col).astype(jnp.float32)
    statT = jnp.dot(ssembT_ref[...], ohT,
                    preferred_element_type=jnp.float32)

    gT = jnp.transpose(g_ref[...])
    ekpT = gT[0:_K, :]
    ekpT_ref[...] = ekpT
    discT = gT[_DISC_COL:_DISC_COL + 1, :]
    w = jax.lax.bitcast_convert_type(
        gT[_SKD_COL:_SKD_COL + _KP2, :], jnp.uint32)
    lo = jax.lax.bitcast_convert_type(
        (w & 0xFFFF).astype(jnp.uint16), jnp.bfloat16)
    hi = jax.lax.bitcast_convert_type(
        (w >> 16).astype(jnp.uint16), jnp.bfloat16)
    skdT = jnp.concatenate([lo, hi], axis=0)[0:_K, :]

    bf = jnp.bfloat16
    xT = (discT.astype(bf) * (statT.astype(bf) - skdT)
          * (mT_ref[...].astype(bf) * _sig(ekpT).astype(bf)))
    h1T = _sig(
        jnp.dot(w1_ref[...].astype(jnp.bfloat16), xT,
                preferred_element_type=jnp.float32)
        + b1_ref[...])
    h2T = _sig(
        jnp.dot(w2_ref[...].astype(jnp.bfloat16), h1T.astype(jnp.bfloat16),
                preferred_element_type=jnp.float32)
        + b2_ref[...])
    pT = _sig(
        jnp.dot(w3_ref[...], h2T, preferred_element_type=jnp.float32)
        + b3_ref[...])
    outT_ref[0:1, :] = 1.0 - pT
    outT_ref[1:2, :] = pT


def _tc_mlp(gathered, masksT, sid, ssembT, w1, b1c, w2, b2c, w3, b3c,
            half_blocks, half):
    BB = 2048
    H = half_blocks * BB
    off = half * half_blocks
    k, stu_n = ssembT.shape
    l1, l2 = w1.shape[0], w2.shape[0]
    full = lambda shp: pl.BlockSpec(shp, lambda i: (0, 0))
    return pl.pallas_call(
        _mlp_body,
        grid=(half_blocks,),
        in_specs=[
            pl.BlockSpec((BB, _D), lambda i: (i, 0)),
            pl.BlockSpec((k, BB), lambda i: (0, i + off)),
            pl.BlockSpec((BB,), lambda i: (i + off,)),
            full((k, stu_n)),
            full((l1, k)), full((l1, 1)),
            full((l2, l1)), full((l2, 1)),
            full((1, l2)), full((1, 1)),
        ],
        out_specs=[
            pl.BlockSpec((2, BB), lambda i: (0, i)),
            pl.BlockSpec((k, BB), lambda i: (0, i)),
        ],
        out_shape=[
            jax.ShapeDtypeStruct((2, H), jnp.float32),
            jax.ShapeDtypeStruct((k, H), jnp.float32),
        ],
    )(gathered, masksT, sid, ssembT, w1, b1c, w2, b2c, w3, b3c)


def kernel(stu_id, input_exercise, knowledge_masks, student_emb, k_difficulty,
           e_discrimination, e_k_prob, W1, b1, W2, b2, W3, b3):
    tbl, ssembT = _prep(k_difficulty.T, e_k_prob.T, e_discrimination.T,
                        student_emb.T)
    idx = input_exercise.astype(jnp.int32)
    gathered = _sc_gather(tbl, idx)
    outT, ekpT = _tc_mlp(
        gathered, knowledge_masks.T, stu_id.astype(jnp.int32),
        ssembT, W1, b1.reshape(-1, 1), W2, b2.reshape(-1, 1),
        W3, b3.reshape(-1, 1), half_blocks=idx.shape[0] // 2048, half=0)
    return (outT.T, ekpT.T)


# speedup vs baseline: 1.3275x; 1.0042x over previous
import functools

import jax
import jax.numpy as jnp
from jax import lax
from jax.experimental import pallas as pl
from jax.experimental.pallas import tpu as pltpu
from jax.experimental.pallas import tpu_sc as plsc

_K = 197
_KP2 = 99
_SKD_COL = 200
_DISC_COL = 304
_D = 384
_NW = 32
_CH = 64


def _sig(x):
    return 0.5 * jnp.tanh(0.5 * x) + 0.5


def _prep_body(kdT_ref, ekpT_ref, discT_ref, sembT_ref, tbl_ref, ssembT_ref):
    n = kdT_ref.shape[1]
    tbl_ref[:, 0:_K] = jnp.transpose(ekpT_ref[...])
    skd = jnp.transpose(_sig(kdT_ref[...])).astype(jnp.bfloat16)
    skd = jnp.concatenate(
        [skd, jnp.zeros((n, 2 * _KP2 - _K), jnp.bfloat16)], axis=1)
    u = jax.lax.bitcast_convert_type(skd, jnp.uint16).astype(jnp.uint32)
    w = u[:, 0:_KP2] | (u[:, _KP2:2 * _KP2] << 16)
    tbl_ref[:, _SKD_COL:_SKD_COL + _KP2] = jax.lax.bitcast_convert_type(
        w, jnp.float32)
    tbl_ref[:, _DISC_COL:_DISC_COL + 1] = jnp.transpose(
        10.0 * _sig(discT_ref[...]))
    ssembT_ref[...] = _sig(sembT_ref[...])


def _prep(kdT, ekpT, discT, sembT):
    k, exer_n = kdT.shape
    stu_n = sembT.shape[1]
    nb = 1024
    grid = ((exer_n + nb - 1) // nb,)
    return pl.pallas_call(
        _prep_body,
        grid=grid,
        in_specs=[
            pl.BlockSpec((k, nb), lambda i: (0, i)),
            pl.BlockSpec((k, nb), lambda i: (0, i)),
            pl.BlockSpec((1, nb), lambda i: (0, i)),
            pl.BlockSpec((k, stu_n), lambda i: (0, 0)),
        ],
        out_specs=[
            pl.BlockSpec((nb, _D), lambda i: (i, 0)),
            pl.BlockSpec((k, stu_n), lambda i: (0, 0)),
        ],
        out_shape=[
            jax.ShapeDtypeStruct((exer_n, _D), jnp.float32),
            jax.ShapeDtypeStruct((k, stu_n), jnp.float32),
        ],
    )(kdT, ekpT, discT, sembT)


def _sc_gather(tbl, idx):
    B = idx.shape[0]
    D = tbl.shape[1]
    bpw = B // _NW
    mesh = plsc.VectorSubcoreMesh(core_axis_name="c", subcore_axis_name="s")

    nch = bpw // _CH

    @functools.partial(
        pl.kernel,
        mesh=mesh,
        out_type=jax.ShapeDtypeStruct((B, D), jnp.float32),
        scratch_types=[
            pltpu.VMEM((_CH,), jnp.int32),
            pltpu.VMEM((_CH,), jnp.int32),
            pltpu.VMEM((_CH,), jnp.int32),
            pltpu.VMEM((_CH, D), jnp.float32),
            pltpu.VMEM((_CH, D), jnp.float32),
            pltpu.VMEM((_CH, D), jnp.float32),
            pltpu.SemaphoreType.DMA,
            pltpu.SemaphoreType.DMA,
            pltpu.SemaphoreType.DMA,
            pltpu.SemaphoreType.DMA,
            pltpu.SemaphoreType.DMA,
            pltpu.SemaphoreType.DMA,
        ],
    )
    def k(tbl_hbm, idx_hbm, out_hbm, idx_v0, idx_v1, idx_v2, r0, r1, r2,
          g0, g1, g2, s0, s1, s2):
        wid = lax.axis_index("s") * 2 + lax.axis_index("c")
        base = wid * bpw
        nb = 3
        idx_v = [idx_v0, idx_v1, idx_v2]
        rows = [r0, r1, r2]
        gsem = [g0, g1, g2]
        ssem = [s0, s1, s2]
        gh = [None] * nb
        sh = [None] * nb
        for b in range(min(nb, nch)):
            pltpu.sync_copy(idx_hbm.at[pl.ds(base + b * _CH, _CH)], idx_v[b])
            gh[b] = pltpu.async_copy(tbl_hbm.at[idx_v[b]], rows[b], gsem[b])
        for ci in range(nch):
            b = ci % nb
            gh[b].wait()
            sh[b] = pltpu.async_copy(
                rows[b], out_hbm.at[pl.ds(base + ci * _CH, _CH)], ssem[b])
            if ci + nb < nch:
                sh[b].wait()
                pltpu.sync_copy(
                    idx_hbm.at[pl.ds(base + (ci + nb) * _CH, _CH)], idx_v[b])
                gh[b] = pltpu.async_copy(tbl_hbm.at[idx_v[b]], rows[b],
                                         gsem[b])
        for b in range(min(nb, nch)):
            if sh[b] is not None:
                sh[b].wait()

    return k(tbl, idx)


def _mlp_body(g_ref, mT_ref, sid_ref, ssembT_ref,
              w1_ref, b1_ref, w2_ref, b2_ref, w3_ref, b3_ref,
              outT_ref, ekpT_ref):
    bb = g_ref.shape[0]
    stu_n = ssembT_ref.shape[1]
    ids = jnp.reshape(sid_ref[...], (1, bb))
    col = lax.broadcasted_iota(jnp.int32, (stu_n, bb), 0)
    ohT = (ids == col).astype(jnp.bfloat16)
    statT = jnp.dot(ssembT_ref[...].astype(jnp.bfloat16), ohT,
                    preferred_element_type=jnp.float32)

    gT = jnp.transpose(g_ref[...])
    ekpT = gT[0:_K, :]
    ekpT_ref[...] = ekpT
    discT = gT[_DISC_COL:_DISC_COL + 1, :]
    w = jax.lax.bitcast_convert_type(
        gT[_SKD_COL:_SKD_COL + _KP2, :], jnp.uint32)
    lo = jax.lax.bitcast_convert_type(
        (w & 0xFFFF).astype(jnp.uint16), jnp.bfloat16)
    hi = jax.lax.bitcast_convert_type(
        (w >> 16).astype(jnp.uint16), jnp.bfloat16)
    skdT = jnp.concatenate([lo, hi], axis=0)[0:_K, :]

    bf = jnp.bfloat16
    xT = (discT.astype(bf) * (statT.astype(bf) - skdT)
          * (mT_ref[...].astype(bf) * _sig(ekpT).astype(bf)))
    h1T = _sig(
        jnp.dot(w1_ref[...].astype(jnp.bfloat16), xT,
                preferred_element_type=jnp.float32)
        + b1_ref[...])
    h2T = _sig(
        jnp.dot(w2_ref[...].astype(jnp.bfloat16), h1T.astype(jnp.bfloat16),
                preferred_element_type=jnp.float32)
        + b2_ref[...])
    pT = _sig(
        jnp.dot(w3_ref[...], h2T, preferred_element_type=jnp.float32)
        + b3_ref[...])
    outT_ref[0:1, :] = 1.0 - pT
    outT_ref[1:2, :] = pT


def _tc_mlp(gathered, masksT, sid, ssembT, w1, b1c, w2, b2c, w3, b3c,
            half_blocks, half):
    BB = 2048
    H = half_blocks * BB
    off = half * half_blocks
    k, stu_n = ssembT.shape
    l1, l2 = w1.shape[0], w2.shape[0]
    full = lambda shp: pl.BlockSpec(shp, lambda i: (0, 0))
    return pl.pallas_call(
        _mlp_body,
        grid=(half_blocks,),
        in_specs=[
            pl.BlockSpec((BB, _D), lambda i: (i, 0)),
            pl.BlockSpec((k, BB), lambda i: (0, i + off)),
            pl.BlockSpec((BB,), lambda i: (i + off,)),
            full((k, stu_n)),
            full((l1, k)), full((l1, 1)),
            full((l2, l1)), full((l2, 1)),
            full((1, l2)), full((1, 1)),
        ],
        out_specs=[
            pl.BlockSpec((2, BB), lambda i: (0, i)),
            pl.BlockSpec((k, BB), lambda i: (0, i)),
        ],
        out_shape=[
            jax.ShapeDtypeStruct((2, H), jnp.float32),
            jax.ShapeDtypeStruct((k, H), jnp.float32),
        ],
    )(gathered, masksT, sid, ssembT, w1, b1c, w2, b2c, w3, b3c)


def kernel(stu_id, input_exercise, knowledge_masks, student_emb, k_difficulty,
           e_discrimination, e_k_prob, W1, b1, W2, b2, W3, b3):
    tbl, ssembT = _prep(k_difficulty.T, e_k_prob.T, e_discrimination.T,
                        student_emb.T)
    idx = input_exercise.astype(jnp.int32)
    gathered = _sc_gather(tbl, idx)
    outT, ekpT = _tc_mlp(
        gathered, knowledge_masks.T, stu_id.astype(jnp.int32),
        ssembT, W1, b1.reshape(-1, 1), W2, b2.reshape(-1, 1),
        W3, b3.reshape(-1, 1), half_blocks=idx.shape[0] // 2048, half=0)
    return (outT.T, ekpT.T)
